# baseline (speedup 1.0000x reference)
import functools

import jax
import jax.numpy as jnp
from jax import lax
from jax.experimental import pallas as pl
from jax.experimental.pallas import tpu as pltpu
from jax.experimental.pallas import tpu_sc as plsc

B, N, D, E, ED = 2, 10000, 256, 160000, 16
DH = 2 * D

NC, NS = 2, 16
EPT = E // NS
CH = 80
NCH = EPT // CH
RPT = N // NS
ZR = 25

F32 = jnp.float32
BF16 = jnp.bfloat16


def _ln(x, g, b, eps=1e-5):
    m = jnp.mean(x, axis=-1, keepdims=True)
    v = jnp.var(x, axis=-1, keepdims=True)
    return (x - m) * jax.lax.rsqrt(v + eps) * g + b


def _gelu(x):
    return x * 0.5 * (1.0 + lax.erf(x * 0.7071067811865476))


def _hm_body(h_ref, w_ref, b_ref, o_ref):
    x = h_ref[...].astype(BF16)
    y = jnp.dot(x, w_ref[...], preferred_element_type=F32) + b_ref[...]
    o_ref[...] = _gelu(y).astype(BF16)


def _gate_body(ea_ref, w1_ref, b1_ref, w2_ref, b2_ref, o_ref):
    a = jnp.dot(ea_ref[...], w1_ref[...], preferred_element_type=F32)
    t = _gelu(a + b1_ref[...]).astype(BF16)
    g = jnp.dot(t, w2_ref[...], preferred_element_type=F32) + b2_ref[...]
    o_ref[...] = jax.nn.sigmoid(g).astype(BF16)


NBUF = 3


def _sc_body(hm_hbm, gate_hbm, src_hbm, dst_hbm, zb_hbm, zd_hbm, ones_hbm,
             agg_out, deg_out,
             sibuf, dibuf, rows, gbuf, onesv,
             sem_idx, sem_gat, sem_gate, sem_sca, sem_deg,
             agg_s, deg_s):
    c = lax.axis_index("c")
    s = lax.axis_index("s")
    is0 = c == 0
    coff = c * N

    pltpu.sync_copy(ones_hbm, onesv)

    @pl.loop(0, 7)
    def _zero(q):
        pltpu.sync_copy(zb_hbm, agg_s.at[pl.ds(s * RPT + q * CH, CH)])

    pltpu.sync_copy(zb_hbm.at[pl.ds(0, RPT - 7 * CH)],
                    agg_s.at[pl.ds(s * RPT + 7 * CH, RPT - 7 * CH)])

    @pl.loop(0, RPT // ZR)
    def _zerod(q):
        pltpu.sync_copy(zd_hbm, deg_s.at[pl.ds(s * RPT + q * ZR, ZR)])

    def issue_idx(k, q):
        ebase = s * EPT + k * CH
        pltpu.async_copy(src_hbm.at[pl.ds(ebase, CH)], sibuf[q], sem_idx[q])
        pltpu.async_copy(dst_hbm.at[pl.ds(ebase, CH)], dibuf[q], sem_idx[q])

    def fire(k, q, b):
        pltpu.make_async_copy(src_hbm.at[pl.ds(0, CH)], sibuf[q],
                              sem_idx[q]).wait()
        pltpu.make_async_copy(dst_hbm.at[pl.ds(0, CH)], dibuf[q],
                              sem_idx[q]).wait()
        for i in range(CH // 16):
            sl = pl.ds(i * 16, 16)
            sibuf[q][sl] = sibuf[q][sl] + coff
        ebase = s * EPT + k * CH
        pltpu.async_copy(hm_hbm.at[sibuf[q]], rows[b], sem_gat[b])
        pltpu.async_copy(gate_hbm.at[pl.ds(ebase, CH)], gbuf[b], sem_gate[b])

    def drain_sca(b):
        pltpu.make_async_copy(rows[b], agg_s.at[dibuf[0]], sem_sca[b]).wait()

        @pl.when(is0)
        def _():
            pltpu.make_async_copy(onesv, deg_s.at[dibuf[0]],
                                  sem_deg[b]).wait()

    def process(k, q, head, tail1, tail2):
        b = q & 1
        o = 1 - b
        if not head:
            drain_sca(o)
        if not tail1:
            fire(k + 1, (q + 1) % 4, o)
        if not tail2:
            issue_idx(k + 2, (q + 2) % 4)
        pltpu.make_async_copy(hm_hbm.at[sibuf[q]], rows[b],
                              sem_gat[b]).wait()
        pltpu.make_async_copy(gate_hbm.at[pl.ds(0, CH)], gbuf[b],
                              sem_gate[b]).wait()

        @pl.loop(0, CH, unroll=4)
        def _mul(r):
            for j in range(D // 32):
                sl = pl.ds(j * 32, 32)
                rows[b][r, sl] = rows[b][r, sl] * gbuf[b][r, sl]

        pltpu.async_copy(rows[b], agg_s.at[dibuf[q]], sem_sca[b], add=True)

        @pl.when(is0)
        def _():
            pltpu.async_copy(onesv, deg_s.at[dibuf[q]], sem_deg[b], add=True)

    issue_idx(0, 0)
    issue_idx(1, 1)
    fire(0, 0, 0)
    plsc.subcore_barrier()

    process(0, 0, True, False, False)
    process(1, 1, False, False, False)

    @pl.loop(0, (NCH - 5) // 4)
    def _quad(jj):
        for i in range(4):
            process(4 * jj + 2 + i, (2 + i) % 4, False, False, False)

    process(NCH - 3, 2, False, False, False)
    process(NCH - 2, 3, False, False, True)
    process(NCH - 1, 0, False, True, True)

    drain_sca(0)

    plsc.subcore_barrier()

    plsc.subcore_barrier()

    obase = s * RPT
    pltpu.sync_copy(agg_s.at[pl.ds(obase, RPT)],
                    agg_out.at[pl.ds(coff + obase, RPT)])

    @pl.when(is0)
    def _degout():
        pltpu.sync_copy(deg_s.at[pl.ds(obase, RPT)],
                        deg_out.at[pl.ds(obase, RPT)])


def _post_body(h_ref, agg_ref, deg_ref, wa_ref, ba_ref, gm_ref, bm_ref,
               wf1_ref, bf1_ref, wf2_ref, bf2_ref, gf_ref, bfb_ref, o_ref):
    deg = jnp.maximum(deg_ref[:, 0:1].astype(F32), 1.0)
    x = agg_ref[...].astype(F32) / deg
    t = _gelu(jnp.dot(x.astype(BF16), wa_ref[...],
                      preferred_element_type=F32) + ba_ref[...])
    h1 = _ln(h_ref[...] + t, gm_ref[...], bm_ref[...])
    u = _gelu(jnp.dot(h1.astype(BF16), wf1_ref[...],
                      preferred_element_type=F32) + bf1_ref[...])
    v = jnp.dot(u.astype(BF16), wf2_ref[...],
                preferred_element_type=F32) + bf2_ref[...]
    o_ref[...] = _ln(h1 + v, gf_ref[...], bfb_ref[...])


def kernel(h, edge_index, edge_attr, W_msg, b_msg, Wg1, bg1, Wg2, bg2, Wa, ba,
           g_msg, b_ln_msg, Wf1, bf1, Wf2, bf2, g_ffn, b_ln_ffn):
    hf = h.reshape(B * N, D)
    src = edge_index[0].astype(jnp.int32)
    dst = edge_index[1].astype(jnp.int32)

    TM = 400
    gm = (B * N) // TM

    hm = pl.pallas_call(
        _hm_body,
        grid=(gm,),
        in_specs=[
            pl.BlockSpec((TM, D), lambda i: (i, 0)),
            pl.BlockSpec((D, D), lambda i: (0, 0)),
            pl.BlockSpec((1, D), lambda i: (0, 0)),
        ],
        out_specs=pl.BlockSpec((TM, D), lambda i: (i, 0)),
        out_shape=jax.ShapeDtypeStruct((B * N, D), BF16),
    )(hf, W_msg.T.astype(BF16), b_msg.reshape(1, D))

    TE = 2000
    ge = E // TE
    gate = pl.pallas_call(
        _gate_body,
        grid=(ge,),
        in_specs=[
            pl.BlockSpec((TE, ED), lambda i: (i, 0)),
            pl.BlockSpec((ED, D), lambda i: (0, 0)),
            pl.BlockSpec((1, D), lambda i: (0, 0)),
            pl.BlockSpec((D, D), lambda i: (0, 0)),
            pl.BlockSpec((1, D), lambda i: (0, 0)),
        ],
        out_specs=pl.BlockSpec((TE, D), lambda i: (i, 0)),
        out_shape=jax.ShapeDtypeStruct((E, D), BF16),
    )(edge_attr, Wg1.T, bg1.reshape(1, D), Wg2.T.astype(BF16),
      bg2.reshape(1, D))

    mesh = plsc.VectorSubcoreMesh(core_axis_name="c", subcore_axis_name="s")
    zb = jnp.zeros((CH, D), BF16)
    zd = jnp.zeros((ZR, 16), BF16)
    onesb = jnp.ones((CH, 16), BF16)
    agg_flat, deg16 = pl.kernel(
        _sc_body,
        out_type=(
            jax.ShapeDtypeStruct((B * N, D), BF16),
            jax.ShapeDtypeStruct((N, 16), BF16),
        ),
        mesh=mesh,
        scratch_types=[
            [pltpu.VMEM((CH,), jnp.int32) for _ in range(4)],
            [pltpu.VMEM((CH,), jnp.int32) for _ in range(4)],
            [pltpu.VMEM((CH, D), BF16) for _ in range(2)],
            [pltpu.VMEM((CH, D), BF16) for _ in range(2)],
            pltpu.VMEM((CH, 16), BF16),
            [pltpu.SemaphoreType.DMA for _ in range(4)],
            [pltpu.SemaphoreType.DMA for _ in range(2)],
            [pltpu.SemaphoreType.DMA for _ in range(2)],
            [pltpu.SemaphoreType.DMA for _ in range(2)],
            [pltpu.SemaphoreType.DMA for _ in range(2)],
            pltpu.VMEM_SHARED((N, D), BF16),
            pltpu.VMEM_SHARED((N, 16), BF16),
        ],
        compiler_params=pltpu.CompilerParams(use_tc_tiling_on_sc=False),
    )(hm, gate, src, dst, zb, zd, onesb)

    out = pl.pallas_call(
        _post_body,
        grid=(gm,),
        in_specs=[
            pl.BlockSpec((TM, D), lambda i: (i, 0)),
            pl.BlockSpec((TM, D), lambda i: (i, 0)),
            pl.BlockSpec((TM, 16), lambda i: (i % (N // TM), 0)),
            pl.BlockSpec((D, D), lambda i: (0, 0)),
            pl.BlockSpec((1, D), lambda i: (0, 0)),
            pl.BlockSpec((1, D), lambda i: (0, 0)),
            pl.BlockSpec((1, D), lambda i: (0, 0)),
            pl.BlockSpec((D, DH), lambda i: (0, 0)),
            pl.BlockSpec((1, DH), lambda i: (0, 0)),
            pl.BlockSpec((DH, D), lambda i: (0, 0)),
            pl.BlockSpec((1, D), lambda i: (0, 0)),
            pl.BlockSpec((1, D), lambda i: (0, 0)),
            pl.BlockSpec((1, D), lambda i: (0, 0)),
        ],
        out_specs=pl.BlockSpec((TM, D), lambda i: (i, 0)),
        out_shape=jax.ShapeDtypeStruct((B * N, D), F32),
    )(hf, agg_flat, deg16, Wa.T.astype(BF16), ba.reshape(1, D),
      g_msg.reshape(1, D), b_ln_msg.reshape(1, D), Wf1.T.astype(BF16),
      bf1.reshape(1, DH), Wf2.T.astype(BF16), bf2.reshape(1, D),
      g_ffn.reshape(1, D), b_ln_ffn.reshape(1, D))

    return out.reshape(B, N, D)


# speedup vs baseline: 1.3553x; 1.3553x over previous
import jax
import jax.numpy as jnp
from jax import lax
from jax.experimental import pallas as pl
from jax.experimental.pallas import tpu as pltpu
from jax.experimental.pallas import tpu_sc as plsc

B, N, D, E, ED = 2, 10000, 256, 160000, 16
DH = 2 * D

NC, NS = 2, 16
EPT = E // NS
CH = 80
NCH = EPT // CH
RPT = N // NS
ZR = 25

F32 = jnp.float32
BF16 = jnp.bfloat16


def _ln(x, g, b, eps=1e-5):
    m = jnp.mean(x, axis=-1, keepdims=True)
    v = jnp.var(x, axis=-1, keepdims=True)
    return (x - m) * jax.lax.rsqrt(v + eps) * g + b


def _gelu(x):
    return x * 0.5 * (1.0 + lax.erf(x * 0.7071067811865476))


def _hm_body(h_ref, w_ref, b_ref, o_ref):
    x = h_ref[...].astype(BF16)
    y = jnp.dot(x, w_ref[...], preferred_element_type=F32) + b_ref[...]
    o_ref[...] = _gelu(y).astype(BF16)


def _gate_body(ea_ref, w1_ref, b1_ref, w2_ref, b2_ref, o_ref):
    a = jnp.dot(ea_ref[...], w1_ref[...], preferred_element_type=F32)
    t = _gelu(a + b1_ref[...]).astype(BF16)
    g = jnp.dot(t, w2_ref[...], preferred_element_type=F32) + b2_ref[...]
    o_ref[...] = jax.nn.sigmoid(g).astype(BF16)


def _sc_body(hm_hbm, gate_hbm, src_hbm, dst_hbm, zb_hbm, zd_hbm, ones_hbm,
             agg_out, deg_out,
             sibuf, dibuf, rows, gbuf, onesv,
             sem_idx, sem_gat, sem_gate, sem_sca, sem_deg,
             agg_s, deg_s):
    c = lax.axis_index("c")
    s = lax.axis_index("s")
    is0 = c == 0
    coff = c * N

    pltpu.sync_copy(ones_hbm, onesv)

    @pl.loop(0, 7)
    def _zero(q):
        pltpu.sync_copy(zb_hbm, agg_s.at[pl.ds(s * RPT + q * CH, CH)])

    pltpu.sync_copy(zb_hbm.at[pl.ds(0, RPT - 7 * CH)],
                    agg_s.at[pl.ds(s * RPT + 7 * CH, RPT - 7 * CH)])

    @pl.loop(0, RPT // ZR)
    def _zerod(q):
        pltpu.sync_copy(zd_hbm, deg_s.at[pl.ds(s * RPT + q * ZR, ZR)])

    def issue_idx(k, q):
        ebase = s * EPT + k * CH
        pltpu.async_copy(src_hbm.at[pl.ds(ebase, CH)], sibuf[q], sem_idx[q])
        pltpu.async_copy(dst_hbm.at[pl.ds(ebase, CH)], dibuf[q], sem_idx[q])

    def fire(k, q, b):
        pltpu.make_async_copy(src_hbm.at[pl.ds(0, CH)], sibuf[q],
                              sem_idx[q]).wait()
        pltpu.make_async_copy(dst_hbm.at[pl.ds(0, CH)], dibuf[q],
                              sem_idx[q]).wait()
        for i in range(CH // 16):
            sl = pl.ds(i * 16, 16)
            sibuf[q][sl] = sibuf[q][sl] + coff
        ebase = s * EPT + k * CH
        pltpu.async_copy(hm_hbm.at[sibuf[q]], rows[b], sem_gat[b])
        pltpu.async_copy(gate_hbm.at[pl.ds(ebase, CH)], gbuf[b], sem_gate[b])

    def drain_sca(b):
        pltpu.make_async_copy(rows[b], agg_s.at[dibuf[0]], sem_sca[b]).wait()

        @pl.when(is0)
        def _():
            pltpu.make_async_copy(onesv, deg_s.at[dibuf[0]],
                                  sem_deg[b]).wait()

    def process(k, q, head, tail1, tail2):
        b = q & 1
        o = 1 - b
        if not head:
            drain_sca(o)
        if not tail1:
            fire(k + 1, (q + 1) % 4, o)
        if not tail2:
            issue_idx(k + 2, (q + 2) % 4)
        pltpu.make_async_copy(hm_hbm.at[sibuf[q]], rows[b],
                              sem_gat[b]).wait()
        pltpu.make_async_copy(gate_hbm.at[pl.ds(0, CH)], gbuf[b],
                              sem_gate[b]).wait()

        @pl.loop(0, CH)
        def _mul(r):
            for j in range(D // 32):
                sl = pl.ds(j * 32, 32)
                rows[b][r, sl] = rows[b][r, sl] * gbuf[b][r, sl]

        pltpu.async_copy(rows[b], agg_s.at[dibuf[q]], sem_sca[b], add=True)

        @pl.when(is0)
        def _():
            pltpu.async_copy(onesv, deg_s.at[dibuf[q]], sem_deg[b], add=True)

    issue_idx(0, 0)
    issue_idx(1, 1)
    fire(0, 0, 0)
    plsc.subcore_barrier()

    process(0, 0, True, False, False)
    process(1, 1, False, False, False)

    @pl.loop(0, (NCH - 5) // 4)
    def _quad(jj):
        for i in range(4):
            process(4 * jj + 2 + i, (2 + i) % 4, False, False, False)

    process(NCH - 3, 2, False, False, False)
    process(NCH - 2, 3, False, False, True)
    process(NCH - 1, 0, False, True, True)

    drain_sca(0)

    plsc.subcore_barrier()

    plsc.subcore_barrier()

    obase = s * RPT
    pltpu.sync_copy(agg_s.at[pl.ds(obase, RPT)],
                    agg_out.at[pl.ds(coff + obase, RPT)])

    @pl.when(is0)
    def _degout():
        pltpu.sync_copy(deg_s.at[pl.ds(obase, RPT)],
                        deg_out.at[pl.ds(obase, RPT)])


def _post_body(h_ref, agg_ref, deg_ref, wa_ref, ba_ref, gm_ref, bm_ref,
               wf1_ref, bf1_ref, wf2_ref, bf2_ref, gf_ref, bfb_ref, o_ref):
    deg = jnp.maximum(deg_ref[:, 0:1].astype(F32), 1.0)
    x = agg_ref[...].astype(F32) / deg
    t = _gelu(jnp.dot(x.astype(BF16), wa_ref[...],
                      preferred_element_type=F32) + ba_ref[...])
    h1 = _ln(h_ref[...] + t, gm_ref[...], bm_ref[...])
    u = _gelu(jnp.dot(h1.astype(BF16), wf1_ref[...],
                      preferred_element_type=F32) + bf1_ref[...])
    v = jnp.dot(u.astype(BF16), wf2_ref[...],
                preferred_element_type=F32) + bf2_ref[...]
    o_ref[...] = _ln(h1 + v, gf_ref[...], bfb_ref[...])


def kernel(h, edge_index, edge_attr, W_msg, b_msg, Wg1, bg1, Wg2, bg2, Wa, ba,
           g_msg, b_ln_msg, Wf1, bf1, Wf2, bf2, g_ffn, b_ln_ffn):
    hf = h.reshape(B * N, D)
    src = edge_index[0].astype(jnp.int32)
    dst = edge_index[1].astype(jnp.int32)

    TM = 400
    gm = (B * N) // TM

    hm = pl.pallas_call(
        _hm_body,
        grid=(gm,),
        in_specs=[
            pl.BlockSpec((TM, D), lambda i: (i, 0)),
            pl.BlockSpec((D, D), lambda i: (0, 0)),
            pl.BlockSpec((1, D), lambda i: (0, 0)),
        ],
        out_specs=pl.BlockSpec((TM, D), lambda i: (i, 0)),
        out_shape=jax.ShapeDtypeStruct((B * N, D), BF16),
    )(hf, W_msg.T.astype(BF16), b_msg.reshape(1, D))

    TE = 2000
    ge = E // TE
    gate = pl.pallas_call(
        _gate_body,
        grid=(ge,),
        in_specs=[
            pl.BlockSpec((TE, ED), lambda i: (i, 0)),
            pl.BlockSpec((ED, D), lambda i: (0, 0)),
            pl.BlockSpec((1, D), lambda i: (0, 0)),
            pl.BlockSpec((D, D), lambda i: (0, 0)),
            pl.BlockSpec((1, D), lambda i: (0, 0)),
        ],
        out_specs=pl.BlockSpec((TE, D), lambda i: (i, 0)),
        out_shape=jax.ShapeDtypeStruct((E, D), BF16),
    )(edge_attr, Wg1.T, bg1.reshape(1, D), Wg2.T.astype(BF16),
      bg2.reshape(1, D))

    mesh = plsc.VectorSubcoreMesh(core_axis_name="c", subcore_axis_name="s")
    zb = jnp.zeros((CH, D), BF16)
    zd = jnp.zeros((ZR, 16), BF16)
    onesb = jnp.ones((CH, 16), BF16)
    agg_flat, deg16 = pl.kernel(
        _sc_body,
        out_type=(
            jax.ShapeDtypeStruct((B * N, D), BF16),
            jax.ShapeDtypeStruct((N, 16), BF16),
        ),
        mesh=mesh,
        scratch_types=[
            [pltpu.VMEM((CH,), jnp.int32) for _ in range(4)],
            [pltpu.VMEM((CH,), jnp.int32) for _ in range(4)],
            [pltpu.VMEM((CH, D), BF16) for _ in range(2)],
            [pltpu.VMEM((CH, D), BF16) for _ in range(2)],
            pltpu.VMEM((CH, 16), BF16),
            [pltpu.SemaphoreType.DMA for _ in range(4)],
            [pltpu.SemaphoreType.DMA for _ in range(2)],
            [pltpu.SemaphoreType.DMA for _ in range(2)],
            [pltpu.SemaphoreType.DMA for _ in range(2)],
            [pltpu.SemaphoreType.DMA for _ in range(2)],
            pltpu.VMEM_SHARED((N, D), BF16),
            pltpu.VMEM_SHARED((N, 16), BF16),
        ],
        compiler_params=pltpu.CompilerParams(use_tc_tiling_on_sc=False),
    )(hm, gate, src, dst, zb, zd, onesb)

    out = pl.pallas_call(
        _post_body,
        grid=(gm,),
        in_specs=[
            pl.BlockSpec((TM, D), lambda i: (i, 0)),
            pl.BlockSpec((TM, D), lambda i: (i, 0)),
            pl.BlockSpec((TM, 16), lambda i: (i % (N // TM), 0)),
            pl.BlockSpec((D, D), lambda i: (0, 0)),
            pl.BlockSpec((1, D), lambda i: (0, 0)),
            pl.BlockSpec((1, D), lambda i: (0, 0)),
            pl.BlockSpec((1, D), lambda i: (0, 0)),
            pl.BlockSpec((D, DH), lambda i: (0, 0)),
            pl.BlockSpec((1, DH), lambda i: (0, 0)),
            pl.BlockSpec((DH, D), lambda i: (0, 0)),
            pl.BlockSpec((1, D), lambda i: (0, 0)),
            pl.BlockSpec((1, D), lambda i: (0, 0)),
            pl.BlockSpec((1, D), lambda i: (0, 0)),
        ],
        out_specs=pl.BlockSpec((TM, D), lambda i: (i, 0)),
        out_shape=jax.ShapeDtypeStruct((B * N, D), F32),
    )(hf, agg_flat, deg16, Wa.T.astype(BF16), ba.reshape(1, D),
      g_msg.reshape(1, D), b_ln_msg.reshape(1, D), Wf1.T.astype(BF16),
      bf1.reshape(1, DH), Wf2.T.astype(BF16), bf2.reshape(1, D),
      g_ffn.reshape(1, D), b_ln_ffn.reshape(1, D))

    return out.reshape(B, N, D)


# speedup vs baseline: 1.3561x; 1.0006x over previous
import jax
import jax.numpy as jnp
from jax import lax
from jax.experimental import pallas as pl
from jax.experimental.pallas import tpu as pltpu
from jax.experimental.pallas import tpu_sc as plsc

B, N, D, E, ED = 2, 10000, 256, 160000, 16
DH = 2 * D

NC, NS = 2, 16
EPT = E // NS
CH = 80
NCH = EPT // CH
RPT = N // NS
ZR = 25

F32 = jnp.float32
BF16 = jnp.bfloat16


def _ln(x, g, b, eps=1e-5):
    m = jnp.mean(x, axis=-1, keepdims=True)
    v = jnp.var(x, axis=-1, keepdims=True)
    return (x - m) * jax.lax.rsqrt(v + eps) * g + b


def _gelu(x):
    return x * 0.5 * (1.0 + lax.erf(x * 0.7071067811865476))


def _hm_body(h_ref, w_ref, b_ref, o_ref):
    x = h_ref[...].astype(BF16)
    y = jnp.dot(x, w_ref[...], preferred_element_type=F32) + b_ref[...]
    o_ref[...] = _gelu(y).astype(BF16)


def _gate_body(ea_ref, w1_ref, b1_ref, w2_ref, b2_ref, o_ref):
    a = jnp.dot(ea_ref[...], w1_ref[...], preferred_element_type=F32)
    t = _gelu(a + b1_ref[...]).astype(BF16)
    g = jnp.dot(t, w2_ref[...], preferred_element_type=F32) + b2_ref[...]
    o_ref[...] = jax.nn.sigmoid(g).astype(BF16)


def _sc_body(hm_hbm, gate_hbm, src_hbm, dst_hbm, zb_hbm, zd_hbm, ones_hbm,
             agg_out, deg_out,
             sibuf, dibuf, rows, gbuf, onesv,
             sem_idx, sem_gat, sem_gate, sem_sca, sem_deg,
             agg_s, deg_s):
    c = lax.axis_index("c")
    s = lax.axis_index("s")
    is0 = c == 0
    coff = c * N

    pltpu.sync_copy(ones_hbm, onesv)

    @pl.loop(0, 7)
    def _zero(q):
        pltpu.sync_copy(zb_hbm, agg_s.at[pl.ds(s * RPT + q * CH, CH)])

    pltpu.sync_copy(zb_hbm.at[pl.ds(0, RPT - 7 * CH)],
                    agg_s.at[pl.ds(s * RPT + 7 * CH, RPT - 7 * CH)])

    @pl.loop(0, RPT // ZR)
    def _zerod(q):
        pltpu.sync_copy(zd_hbm, deg_s.at[pl.ds(s * RPT + q * ZR, ZR)])

    def issue_idx(k, q):
        ebase = s * EPT + k * CH
        pltpu.async_copy(src_hbm.at[pl.ds(ebase, CH)], sibuf[q], sem_idx[q])
        pltpu.async_copy(dst_hbm.at[pl.ds(ebase, CH)], dibuf[q], sem_idx[q])

    def fire(k, q, b):
        pltpu.make_async_copy(src_hbm.at[pl.ds(0, CH)], sibuf[q],
                              sem_idx[q]).wait()
        pltpu.make_async_copy(dst_hbm.at[pl.ds(0, CH)], dibuf[q],
                              sem_idx[q]).wait()
        for i in range(CH // 16):
            sl = pl.ds(i * 16, 16)
            sibuf[q][sl] = sibuf[q][sl] + coff
        ebase = s * EPT + k * CH
        pltpu.async_copy(hm_hbm.at[sibuf[q]], rows[b], sem_gat[b])
        pltpu.async_copy(gate_hbm.at[pl.ds(ebase, CH)], gbuf[b], sem_gate[b])

    def drain_sca(b):
        pltpu.make_async_copy(rows[b], agg_s.at[dibuf[0]], sem_sca[b]).wait()

        @pl.when(is0)
        def _():
            pltpu.make_async_copy(onesv, deg_s.at[dibuf[0]],
                                  sem_deg[b]).wait()

    def process(k, q, head, tail1, tail2):
        b = q & 1
        o = 1 - b
        if not head:
            drain_sca(o)
        if not tail1:
            fire(k + 1, (q + 1) % 4, o)
        if not tail2:
            issue_idx(k + 2, (q + 2) % 4)
        pltpu.make_async_copy(hm_hbm.at[sibuf[q]], rows[b],
                              sem_gat[b]).wait()
        pltpu.make_async_copy(gate_hbm.at[pl.ds(0, CH)], gbuf[b],
                              sem_gate[b]).wait()

        @pl.loop(0, CH)
        def _mul(r):
            for j in range(D // 32):
                sl = pl.ds(j * 32, 32)
                rows[b][r, sl] = rows[b][r, sl] * gbuf[b][r, sl]

        pltpu.async_copy(rows[b], agg_s.at[dibuf[q]], sem_sca[b], add=True)

        @pl.when(is0)
        def _():
            pltpu.async_copy(onesv, deg_s.at[dibuf[q]], sem_deg[b], add=True)

    issue_idx(0, 0)
    issue_idx(1, 1)
    fire(0, 0, 0)
    plsc.subcore_barrier()

    process(0, 0, True, False, False)
    process(1, 1, False, False, False)

    @pl.loop(0, (NCH - 5) // 4)
    def _quad(jj):
        for i in range(4):
            process(4 * jj + 2 + i, (2 + i) % 4, False, False, False)

    process(NCH - 3, 2, False, False, False)
    process(NCH - 2, 3, False, False, True)
    process(NCH - 1, 0, False, True, True)

    drain_sca(0)

    plsc.subcore_barrier()

    obase = s * RPT
    pltpu.sync_copy(agg_s.at[pl.ds(obase, RPT)],
                    agg_out.at[pl.ds(coff + obase, RPT)])

    @pl.when(is0)
    def _degout():
        pltpu.sync_copy(deg_s.at[pl.ds(obase, RPT)],
                        deg_out.at[pl.ds(obase, RPT)])


def _post_body(h_ref, agg_ref, deg_ref, wa_ref, ba_ref, gm_ref, bm_ref,
               wf1_ref, bf1_ref, wf2_ref, bf2_ref, gf_ref, bfb_ref, o_ref):
    deg = jnp.maximum(deg_ref[:, 0:1].astype(F32), 1.0)
    x = agg_ref[...].astype(F32) / deg
    t = _gelu(jnp.dot(x.astype(BF16), wa_ref[...],
                      preferred_element_type=F32) + ba_ref[...])
    h1 = _ln(h_ref[...] + t, gm_ref[...], bm_ref[...])
    u = _gelu(jnp.dot(h1.astype(BF16), wf1_ref[...],
                      preferred_element_type=F32) + bf1_ref[...])
    v = jnp.dot(u.astype(BF16), wf2_ref[...],
                preferred_element_type=F32) + bf2_ref[...]
    o_ref[...] = _ln(h1 + v, gf_ref[...], bfb_ref[...])


def kernel(h, edge_index, edge_attr, W_msg, b_msg, Wg1, bg1, Wg2, bg2, Wa, ba,
           g_msg, b_ln_msg, Wf1, bf1, Wf2, bf2, g_ffn, b_ln_ffn):
    hf = h.reshape(B * N, D)
    src = edge_index[0].astype(jnp.int32)
    dst = edge_index[1].astype(jnp.int32)

    TM = 400
    gm = (B * N) // TM

    hm = pl.pallas_call(
        _hm_body,
        grid=(gm,),
        in_specs=[
            pl.BlockSpec((TM, D), lambda i: (i, 0)),
            pl.BlockSpec((D, D), lambda i: (0, 0)),
            pl.BlockSpec((1, D), lambda i: (0, 0)),
        ],
        out_specs=pl.BlockSpec((TM, D), lambda i: (i, 0)),
        out_shape=jax.ShapeDtypeStruct((B * N, D), BF16),
    )(hf, W_msg.T.astype(BF16), b_msg.reshape(1, D))

    TE = 2000
    ge = E // TE
    gate = pl.pallas_call(
        _gate_body,
        grid=(ge,),
        in_specs=[
            pl.BlockSpec((TE, ED), lambda i: (i, 0)),
            pl.BlockSpec((ED, D), lambda i: (0, 0)),
            pl.BlockSpec((1, D), lambda i: (0, 0)),
            pl.BlockSpec((D, D), lambda i: (0, 0)),
            pl.BlockSpec((1, D), lambda i: (0, 0)),
        ],
        out_specs=pl.BlockSpec((TE, D), lambda i: (i, 0)),
        out_shape=jax.ShapeDtypeStruct((E, D), BF16),
    )(edge_attr, Wg1.T, bg1.reshape(1, D), Wg2.T.astype(BF16),
      bg2.reshape(1, D))

    mesh = plsc.VectorSubcoreMesh(core_axis_name="c", subcore_axis_name="s")
    zb = jnp.zeros((CH, D), BF16)
    zd = jnp.zeros((ZR, 16), BF16)
    onesb = jnp.ones((CH, 16), BF16)
    agg_flat, deg16 = pl.kernel(
        _sc_body,
        out_type=(
            jax.ShapeDtypeStruct((B * N, D), BF16),
            jax.ShapeDtypeStruct((N, 16), BF16),
        ),
        mesh=mesh,
        scratch_types=[
            [pltpu.VMEM((CH,), jnp.int32) for _ in range(4)],
            [pltpu.VMEM((CH,), jnp.int32) for _ in range(4)],
            [pltpu.VMEM((CH, D), BF16) for _ in range(2)],
            [pltpu.VMEM((CH, D), BF16) for _ in range(2)],
            pltpu.VMEM((CH, 16), BF16),
            [pltpu.SemaphoreType.DMA for _ in range(4)],
            [pltpu.SemaphoreType.DMA for _ in range(2)],
            [pltpu.SemaphoreType.DMA for _ in range(2)],
            [pltpu.SemaphoreType.DMA for _ in range(2)],
            [pltpu.SemaphoreType.DMA for _ in range(2)],
            pltpu.VMEM_SHARED((N, D), BF16),
            pltpu.VMEM_SHARED((N, 16), BF16),
        ],
        compiler_params=pltpu.CompilerParams(use_tc_tiling_on_sc=False),
    )(hm, gate, src, dst, zb, zd, onesb)

    out = pl.pallas_call(
        _post_body,
        grid=(gm,),
        in_specs=[
            pl.BlockSpec((TM, D), lambda i: (i, 0)),
            pl.BlockSpec((TM, D), lambda i: (i, 0)),
            pl.BlockSpec((TM, 16), lambda i: (i % (N // TM), 0)),
            pl.BlockSpec((D, D), lambda i: (0, 0)),
            pl.BlockSpec((1, D), lambda i: (0, 0)),
            pl.BlockSpec((1, D), lambda i: (0, 0)),
            pl.BlockSpec((1, D), lambda i: (0, 0)),
            pl.BlockSpec((D, DH), lambda i: (0, 0)),
            pl.BlockSpec((1, DH), lambda i: (0, 0)),
            pl.BlockSpec((DH, D), lambda i: (0, 0)),
            pl.BlockSpec((1, D), lambda i: (0, 0)),
            pl.BlockSpec((1, D), lambda i: (0, 0)),
            pl.BlockSpec((1, D), lambda i: (0, 0)),
        ],
        out_specs=pl.BlockSpec((TM, D), lambda i: (i, 0)),
        out_shape=jax.ShapeDtypeStruct((B * N, D), F32),
    )(hf, agg_flat, deg16, Wa.T.astype(BF16), ba.reshape(1, D),
      g_msg.reshape(1, D), b_ln_msg.reshape(1, D), Wf1.T.astype(BF16),
      bf1.reshape(1, DH), Wf2.T.astype(BF16), bf2.reshape(1, D),
      g_ffn.reshape(1, D), b_ln_ffn.reshape(1, D))

    return out.reshape(B, N, D)
